# baseline (speedup 1.0000x reference)
import functools

import jax
import jax.numpy as jnp
from jax import lax
from jax.experimental import pallas as pl
from jax.experimental.pallas import tpu as pltpu
from jax.experimental.pallas import tpu_sc as plsc

N_NODES = 10000
N_PAD = 10240
N_EDGES = 320000
D = 128
N_ITERS = 8
ALPHA = 0.1

NC = 2
NS = 16
NW = NC * NS
EC = 80
NCH = 126
E_PAD = NW * NCH * EC
ROWS_PER_TILE = N_PAD // NS

BM = 1024
GRID_M = N_PAD // BM


NBF = 2
NF = 3
ND = 6


def _propagate_body(z_hbm, src_hbm, dst_hbm, w_hbm, out_hbm,
                    src_b, dst_b, w_b, rb0, rb1, fr0, fr1, fr2,
                    g0, g1, s0, s1, s2, i0, i1, acc):
    rb = (rb0, rb1)
    fr = (fr0, fr1, fr2)
    gsem = (g0, g1)
    ssem = (s0, s1, s2)
    isem = (i0, i1)
    c = lax.axis_index("c")
    s = lax.axis_index("s")
    wid = c * NS + s
    ebase = wid * NCH * EC

    zero16 = jnp.zeros((16,), jnp.float32)

    def _zrow(r, carry):
        for g in range(8):
            fr0[r, pl.ds(g * 16, 16)] = zero16
        return carry

    lax.fori_loop(0, EC, _zrow, 0)
    row0 = s * ROWS_PER_TILE
    for k in range(ROWS_PER_TILE // EC):
        pltpu.sync_copy(fr0, acc.at[pl.ds(row0 + k * EC, EC)])
    plsc.subcore_barrier()

    def _issue_src(j, p):
        pltpu.make_async_copy(
            src_hbm.at[pl.ds(ebase + j * EC, EC)], src_b.at[p], isem[p]).start()

    def _wait_src(j, p):
        pltpu.make_async_copy(
            src_hbm.at[pl.ds(ebase + j * EC, EC)], src_b.at[p], isem[p]).wait()

    def _issue_gather(j, p, d):
        pltpu.make_async_copy(
            z_hbm.at[src_b.at[p]], rb[p], gsem[p]).start()
        pltpu.make_async_copy(
            dst_hbm.at[pl.ds(ebase + j * EC, EC)], dst_b.at[d], gsem[p]).start()
        pltpu.make_async_copy(
            w_hbm.at[pl.ds(ebase + j * EC, EC)], w_b.at[p], gsem[p]).start()

    def _wait_gather(j, p, d):
        pltpu.make_async_copy(
            z_hbm.at[src_b.at[p]], rb[p], gsem[p]).wait()
        pltpu.make_async_copy(
            dst_hbm.at[pl.ds(ebase + j * EC, EC)], dst_b.at[d], gsem[p]).wait()
        pltpu.make_async_copy(
            w_hbm.at[pl.ds(ebase + j * EC, EC)], w_b.at[p], gsem[p]).wait()

    def _scale(k):
        p, b = k % NBF, k % NF
        rb_, fr_ = rb[p], fr[b]

        def _grp(g, carry2):
            wv = w_b[p, pl.ds(g * 16, 16)]
            for l in range(16):
                e = g * 16 + l
                wsc = wv[l]
                for q in range(4):
                    v = rb_[e, pl.ds(q * 16, 16)]
                    av = lax.bitcast_convert_type(v << 16, jnp.float32)
                    bv = lax.bitcast_convert_type(
                        v & jnp.int32(-65536), jnp.float32)
                    fr_[e, pl.ds(q * 32, 16)] = av * wsc
                    fr_[e, pl.ds(q * 32 + 16, 16)] = bv * wsc
            return carry2

        lax.fori_loop(0, EC // 16, _grp, 0)

    for p in range(NBF):
        _issue_src(p, p)
    for p in range(NBF):
        _wait_src(p, p)
    for p in range(NBF):
        _issue_gather(p, p, p)

    def _trip(t, carry):
        j0 = t * ND
        descs = {}
        for k in range(ND):
            j = j0 + k
            p, b, d = k % NBF, k % NF, k % ND
            _wait_gather(j, p, d)

            @pl.when(j + 2 < NCH)
            def _():
                _issue_src(j + 2, p)

            if k >= NF:
                descs[k - NF].wait()
            _scale(k)
            descs[k] = pltpu.async_copy(
                fr[b], acc.at[dst_b.at[d]], ssem[b], add=True)

            @pl.when(j + 2 < NCH)
            def _():
                _wait_src(j + 2, p)
                _issue_gather(j + 2, p, (d + 2) % ND)

        for k in range(NF):
            descs[ND - NF + k].wait()
        return carry

    lax.fori_loop(0, NCH // ND, _trip, 0)
    plsc.subcore_barrier()

    pltpu.sync_copy(acc.at[pl.ds(row0, ROWS_PER_TILE)],
                    out_hbm.at[c].at[pl.ds(row0, ROWS_PER_TILE)])


_propagate = functools.partial(
    pl.kernel,
    out_type=jax.ShapeDtypeStruct((NC, N_PAD, D), jnp.float32),
    mesh=plsc.VectorSubcoreMesh(core_axis_name="c", subcore_axis_name="s"),
    compiler_params=pltpu.CompilerParams(use_tc_tiling_on_sc=False),
    scratch_types=(
        [pltpu.VMEM((NBF, EC), jnp.int32),
         pltpu.VMEM((ND, EC), jnp.int32),
         pltpu.VMEM((NBF, EC), jnp.float32)]
        + [pltpu.VMEM((EC, D // 2), jnp.int32) for _ in range(NBF)]
        + [pltpu.VMEM((EC, D), jnp.float32) for _ in range(NF)]
        + [pltpu.SemaphoreType.DMA for _ in range(NBF + NF + NBF)]
        + [pltpu.VMEM_SHARED((N_PAD, D), jnp.float32)]
    ),
)(_propagate_body)


def _pack_bf16_pairs(zf):
    ua = lax.bitcast_convert_type(zf[:, :64], jnp.uint32)
    ub = lax.bitcast_convert_type(zf[:, 64:], jnp.uint32)
    ra = (ua + jnp.uint32(0x7FFF) + ((ua >> 16) & jnp.uint32(1))) >> 16
    rb = (ub + jnp.uint32(0x7FFF) + ((ub >> 16) & jnp.uint32(1))) >> 16
    return lax.bitcast_convert_type((rb << 16) | ra, jnp.int32)


def _enc_body(x_ref, we_ref, wb_ref, wv_ref, bx_ref, z_ref):
    h = jnp.dot(x_ref[...], we_ref[...], preferred_element_type=jnp.float32)
    bx = jnp.dot(h, wb_ref[...], preferred_element_type=jnp.float32)
    bx_ref[...] = bx
    z_ref[...] = _pack_bf16_pairs(
        jnp.dot(-ALPHA * bx, wv_ref[...],
                preferred_element_type=jnp.float32))


def _iter_body(u_ref, v_ref, bx_ref, wv_ref, un_ref, z_ref):
    u = u_ref[...]
    nu = jnp.maximum(u, 0.0)
    u_new = 2.0 * (v_ref[0] + v_ref[1]) - 2.0 * nu + u
    un_ref[...] = u_new
    nu2 = jnp.maximum(u_new, 0.0)
    z_ref[...] = _pack_bf16_pairs(
        jnp.dot(2.0 * nu2 - u_new - ALPHA * bx_ref[...], wv_ref[...],
                preferred_element_type=jnp.float32))


def _fin_body(u_ref, v_ref, wd_ref, out_ref):
    u = u_ref[...]
    nu = jnp.maximum(u, 0.0)
    u_new = 2.0 * (v_ref[0] + v_ref[1]) - 2.0 * nu + u
    out_ref[...] = jnp.dot(jnp.maximum(u_new, 0.0), wd_ref[...],
                           preferred_element_type=jnp.float32)


_row_spec = pl.BlockSpec((BM, D), lambda i: (i, 0))
_w_spec = pl.BlockSpec((D, D), lambda i: (0, 0))
_v_spec = pl.BlockSpec((NC, BM, D), lambda i: (0, i, 0))
_nd = jax.ShapeDtypeStruct((N_PAD, D), jnp.float32)
_zi_spec = pl.BlockSpec((BM, D // 2), lambda i: (i, 0))
_nd_zi = jax.ShapeDtypeStruct((N_PAD, D // 2), jnp.int32)

_PERM = ([32 * (j // 16) + j % 16 for j in range(64)]
         + [32 * (j // 16) + 16 + j % 16 for j in range(64)])

_encoder = pl.pallas_call(
    _enc_body, grid=(GRID_M,),
    in_specs=[_row_spec, _w_spec, _w_spec, _w_spec],
    out_specs=[_row_spec, _zi_spec],
    out_shape=[_nd, _nd_zi],
)

_iterate = pl.pallas_call(
    _iter_body, grid=(GRID_M,),
    in_specs=[_row_spec, _v_spec, _row_spec, _w_spec],
    out_specs=[_row_spec, _zi_spec],
    out_shape=[_nd, _nd_zi],
)

_finalize = pl.pallas_call(
    _fin_body, grid=(GRID_M,),
    in_specs=[_row_spec, _v_spec, _w_spec],
    out_specs=_row_spec,
    out_shape=_nd,
)


def kernel(x, edge_index, edge_weight, W_enc, W_bias, W_V, W_dec):
    src = edge_index[0].astype(jnp.int32)
    dst = edge_index[1].astype(jnp.int32)
    w = edge_weight.astype(jnp.float32)

    pad = E_PAD - N_EDGES
    fill = jnp.arange(pad, dtype=jnp.int32) % N_NODES
    src_p = jnp.concatenate([src, fill])
    dst_p = jnp.concatenate([dst, fill])
    w_p = jnp.concatenate([w, jnp.zeros((pad,), jnp.float32)])

    x_p = jnp.pad(x, ((0, N_PAD - N_NODES), (0, 0)))
    wv_t_perm = W_V.T[:, jnp.array(_PERM, dtype=jnp.int32)]
    bx, z = _encoder(x_p, W_enc.T, W_bias.T, wv_t_perm)
    u = jnp.zeros_like(bx)
    for i in range(N_ITERS):
        v = _propagate(z, src_p, dst_p, w_p)
        if i < N_ITERS - 1:
            u, z = _iterate(u, v, bx, wv_t_perm)
        else:
            out = _finalize(u, v, W_dec.T)
    return out[:N_NODES]


# speedup vs baseline: 1.8451x; 1.8451x over previous
import functools

import jax
import jax.numpy as jnp
from jax import lax
from jax.experimental import pallas as pl
from jax.experimental.pallas import tpu as pltpu
from jax.experimental.pallas import tpu_sc as plsc

N_NODES = 10000
N_PAD = 10240
N_EDGES = 320000
D = 128
N_ITERS = 8
ALPHA = 0.1

NC = 2
NS = 16
NW = NC * NS
EC = 80
NCH = 128
E_PAD = NW * NCH * EC
ROWS_PER_TILE = N_PAD // NS

BM = 1024
GRID_M = N_PAD // BM


NBUF = 4


def _propagate_body(z_hbm, src_hbm, dst_hbm, w_hbm, out_hbm,
                    src_b, dst_b, w_b,
                    rows0, rows1, rows2, rows3,
                    g0, g1, g2, g3, s0, s1, s2, s3, i0, i1, i2, i3,
                    acc):
    rows = (rows0, rows1, rows2, rows3)
    gsem = (g0, g1, g2, g3)
    ssem = (s0, s1, s2, s3)
    isem = (i0, i1, i2, i3)
    c = lax.axis_index("c")
    s = lax.axis_index("s")
    wid = c * NS + s
    ebase = wid * NCH * EC

    zero16 = jnp.zeros((16,), jnp.float32)

    def _zrow(r, carry):
        for g in range(8):
            rows0[r, pl.ds(g * 16, 16)] = zero16
        return carry

    lax.fori_loop(0, EC, _zrow, 0)
    row0 = s * ROWS_PER_TILE
    for k in range(ROWS_PER_TILE // EC):
        pltpu.sync_copy(rows0, acc.at[pl.ds(row0 + k * EC, EC)])
    plsc.subcore_barrier()

    def _issue_src(j, b):
        pltpu.make_async_copy(
            src_hbm.at[pl.ds(ebase + j * EC, EC)], src_b.at[b], isem[b]).start()

    def _wait_src(j, b):
        pltpu.make_async_copy(
            src_hbm.at[pl.ds(ebase + j * EC, EC)], src_b.at[b], isem[b]).wait()

    def _issue_gather(j, b):
        pltpu.make_async_copy(
            z_hbm.at[src_b.at[b]], rows[b], gsem[b]).start()
        pltpu.make_async_copy(
            dst_hbm.at[pl.ds(ebase + j * EC, EC)], dst_b.at[b], gsem[b]).start()
        pltpu.make_async_copy(
            w_hbm.at[pl.ds(ebase + j * EC, EC)], w_b.at[b], gsem[b]).start()

    def _wait_gather(j, b):
        pltpu.make_async_copy(
            z_hbm.at[src_b.at[b]], rows[b], gsem[b]).wait()
        pltpu.make_async_copy(
            dst_hbm.at[pl.ds(ebase + j * EC, EC)], dst_b.at[b], gsem[b]).wait()
        pltpu.make_async_copy(
            w_hbm.at[pl.ds(ebase + j * EC, EC)], w_b.at[b], gsem[b]).wait()

    def _scale(buf, b):
        def _grp(g, carry2):
            wv = w_b[b, pl.ds(g * 16, 16)]
            for l in range(16):
                e = g * 16 + l
                wsc = wv[l]
                for q in range(8):
                    buf[e, pl.ds(q * 16, 16)] = buf[e, pl.ds(q * 16, 16)] * wsc
            return carry2

        lax.fori_loop(0, EC // 16, _grp, 0)

    for b in range(NBUF):
        _issue_src(b, b)
    for b in range(NBUF):
        _wait_src(b, b)
    for b in range(NBUF):
        _issue_gather(b, b)

    def _trip(jj, carry):
        descs = []
        for b in range(NBUF):
            j = jj * NBUF + b
            _wait_gather(j, b)

            @pl.when(j + NBUF < NCH)
            def _():
                _issue_src(j + NBUF, b)

            _scale(rows[b], b)
            descs.append(pltpu.async_copy(
                rows[b], acc.at[dst_b.at[b]], ssem[b], add=True))
        for b in range(NBUF):
            descs[b].wait()
            nj = (jj + 1) * NBUF + b

            @pl.when(nj < NCH)
            def _():
                _wait_src(nj, b)
                _issue_gather(nj, b)

        return carry

    lax.fori_loop(0, NCH // NBUF, _trip, 0)
    plsc.subcore_barrier()

    pltpu.sync_copy(acc.at[pl.ds(row0, ROWS_PER_TILE)],
                    out_hbm.at[c].at[pl.ds(row0, ROWS_PER_TILE)])


_propagate = functools.partial(
    pl.kernel,
    out_type=jax.ShapeDtypeStruct((NC, N_PAD, D), jnp.float32),
    mesh=plsc.VectorSubcoreMesh(core_axis_name="c", subcore_axis_name="s"),
    scratch_types=(
        [pltpu.VMEM((NBUF, EC), jnp.int32),
         pltpu.VMEM((NBUF, EC), jnp.int32),
         pltpu.VMEM((NBUF, EC), jnp.float32)]
        + [pltpu.VMEM((EC, D), jnp.float32) for _ in range(NBUF)]
        + [pltpu.SemaphoreType.DMA for _ in range(3 * NBUF)]
        + [pltpu.VMEM_SHARED((N_PAD, D), jnp.float32)]
    ),
)(_propagate_body)


def _enc_body(x_ref, we_ref, wb_ref, wv_ref, bx_ref, z_ref):
    h = jnp.dot(x_ref[...], we_ref[...], preferred_element_type=jnp.float32)
    bx = jnp.dot(h, wb_ref[...], preferred_element_type=jnp.float32)
    bx_ref[...] = bx
    z_ref[...] = jnp.dot(-ALPHA * bx, wv_ref[...],
                         preferred_element_type=jnp.float32)


def _iter_body(u_ref, v_ref, bx_ref, wv_ref, un_ref, z_ref):
    u = u_ref[...]
    nu = jnp.maximum(u, 0.0)
    u_new = 2.0 * (v_ref[0] + v_ref[1]) - 2.0 * nu + u
    un_ref[...] = u_new
    nu2 = jnp.maximum(u_new, 0.0)
    z_ref[...] = jnp.dot(2.0 * nu2 - u_new - ALPHA * bx_ref[...], wv_ref[...],
                         preferred_element_type=jnp.float32)


def _fin_body(u_ref, v_ref, wd_ref, out_ref):
    u = u_ref[...]
    nu = jnp.maximum(u, 0.0)
    u_new = 2.0 * (v_ref[0] + v_ref[1]) - 2.0 * nu + u
    out_ref[...] = jnp.dot(jnp.maximum(u_new, 0.0), wd_ref[...],
                           preferred_element_type=jnp.float32)


_row_spec = pl.BlockSpec((BM, D), lambda i: (i, 0))
_w_spec = pl.BlockSpec((D, D), lambda i: (0, 0))
_v_spec = pl.BlockSpec((NC, BM, D), lambda i: (0, i, 0))
_nd = jax.ShapeDtypeStruct((N_PAD, D), jnp.float32)

_encoder = pl.pallas_call(
    _enc_body, grid=(GRID_M,),
    in_specs=[_row_spec, _w_spec, _w_spec, _w_spec],
    out_specs=[_row_spec, _row_spec],
    out_shape=[_nd, _nd],
)

_iterate = pl.pallas_call(
    _iter_body, grid=(GRID_M,),
    in_specs=[_row_spec, _v_spec, _row_spec, _w_spec],
    out_specs=[_row_spec, _row_spec],
    out_shape=[_nd, _nd],
)

_finalize = pl.pallas_call(
    _fin_body, grid=(GRID_M,),
    in_specs=[_row_spec, _v_spec, _w_spec],
    out_specs=_row_spec,
    out_shape=_nd,
)


def kernel(x, edge_index, edge_weight, W_enc, W_bias, W_V, W_dec):
    src = edge_index[0].astype(jnp.int32)
    dst = edge_index[1].astype(jnp.int32)
    w = edge_weight.astype(jnp.float32)

    pad = E_PAD - N_EDGES
    fill = jnp.arange(pad, dtype=jnp.int32) % N_NODES
    src_p = jnp.concatenate([src, fill])
    dst_p = jnp.concatenate([dst, fill])
    w_p = jnp.concatenate([w, jnp.zeros((pad,), jnp.float32)])

    x_p = jnp.pad(x, ((0, N_PAD - N_NODES), (0, 0)))
    bx, z = _encoder(x_p, W_enc.T, W_bias.T, W_V.T)
    u = jnp.zeros_like(bx)
    for i in range(N_ITERS):
        v = _propagate(z, src_p, dst_p, w_p)
        if i < N_ITERS - 1:
            u, z = _iterate(u, v, bx, W_V.T)
        else:
            out = _finalize(u, v, W_dec.T)
    return out[:N_NODES]
